# Initial kernel scaffold; baseline (speedup 1.0000x reference)
import jax
import jax.numpy as jnp
from jax.experimental import pallas as pl


def kernel(point_seq):
    raise NotImplementedError("write your pallas kernel here")


# speedup vs baseline: 25.5326x; 25.5326x over previous
import dataclasses
from functools import partial

import jax
import jax.numpy as jnp
from jax import lax
from jax.experimental import pallas as pl
from jax.experimental.pallas import tpu as pltpu
from jax.experimental.pallas import tpu_sc as plsc

_K = 16
_STRIDE = 8
_ROWS = 128


def _knn_body(n_extract, want_di, keys_ref, q0_ref, *refs):
    if want_di:
        dist_ref, idx_ref, outx_ref, qcur_ref = refs
    else:
        outx_ref, qcur_ref = refs
    s = pl.program_id(2)
    nkeys = keys_ref.shape[3]

    @pl.when(s == 0)
    def _():
        qcur_ref[...] = q0_ref[0]

    q = qcur_ref[...]
    qx, qy, qz = q[:, 0:1], q[:, 1:2], q[:, 2:3]
    kx = keys_ref[0, 0, 0:1, :]
    ky = keys_ref[0, 0, 1:2, :]
    kz = keys_ref[0, 0, 2:3, :]
    dx = qx - kx
    dy = qy - ky
    dz = qz - kz
    dd = (dx * dx + dy * dy) + dz * dz
    iota = lax.broadcasted_iota(jnp.int32, dd.shape, 1)
    ms, mis = [], []
    for e in range(n_extract):
        m = jnp.min(dd, axis=1, keepdims=True)
        mi = jnp.min(jnp.where(dd == m, iota, jnp.int32(nkeys)), axis=1,
                     keepdims=True)
        onehot = iota == mi
        if e == 0:
            ohf = onehot.astype(jnp.float32)
            nx = jnp.sum(ohf * kx, axis=1, keepdims=True)
            ny = jnp.sum(ohf * ky, axis=1, keepdims=True)
            nz = jnp.sum(ohf * kz, axis=1, keepdims=True)
            new_q = jnp.concatenate([nx, ny, nz], axis=1)
            qcur_ref[...] = new_q
            outx_ref[0, 0] = new_q
        if e < n_extract - 1:
            dd = jnp.where(onehot, jnp.float32(jnp.inf), dd)
        ms.append(m)
        mis.append(mi)
    if want_di:
        dist_ref[0, 0] = jnp.concatenate(ms, axis=1)
        idx_ref[0, 0] = jnp.concatenate(mis, axis=1)


def _knn_call(keys, q0, n_extract, want_di, interpret=False):
    c, t, _, n = keys.shape
    r = q0.shape[1]
    grid = (c, r // _ROWS, t)
    out_shapes = []
    out_specs = []
    if want_di:
        out_shapes += [jax.ShapeDtypeStruct((c, t, r, _K), jnp.float32),
                       jax.ShapeDtypeStruct((c, t, r, _K), jnp.int32)]
        out_specs += [pl.BlockSpec((1, 1, _ROWS, _K),
                                   lambda ci, ri, si: (ci, si, ri, 0))] * 2
    out_shapes.append(jax.ShapeDtypeStruct((c, t, r, 3), jnp.float32))
    out_specs.append(pl.BlockSpec((1, 1, _ROWS, 3),
                                  lambda ci, ri, si: (ci, si, ri, 0)))
    return pl.pallas_call(
        partial(_knn_body, n_extract, want_di),
        grid=grid,
        in_specs=[
            pl.BlockSpec((1, 1, 3, n), lambda ci, ri, si: (ci, si, 0, 0)),
            pl.BlockSpec((1, _ROWS, 3), lambda ci, ri, si: (ci, ri, 0)),
        ],
        out_specs=out_specs,
        out_shape=out_shapes,
        scratch_shapes=[pltpu.VMEM((_ROWS, 3), jnp.float32)],
        interpret=interpret,
    )(keys, q0)


def _sc_gather(tab, idx):
    g_tot, m = idx.shape
    nw = 32
    gpw = g_tot // nw
    mesh = plsc.VectorSubcoreMesh(core_axis_name="c", subcore_axis_name="s")
    cp = pltpu.CompilerParams()
    if "needs_layout_passes" in pltpu.CompilerParams.__dataclass_fields__:
        cp = dataclasses.replace(cp, needs_layout_passes=False)

    @partial(pl.kernel,
             out_type=jax.ShapeDtypeStruct((g_tot, 6, m), jnp.float32),
             mesh=mesh,
             scratch_types=[pltpu.VMEM((tab.shape[1],), jnp.float32),
                            pltpu.VMEM((m,), jnp.int32),
                            pltpu.VMEM((6, m), jnp.float32)],
             compiler_params=cp)
    def k(tab_hbm, idx_hbm, out_hbm, tab_v, idx_v, out_v):
        wid = lax.axis_index("s") * 2 + lax.axis_index("c")

        @pl.loop(0, gpw)
        def _(gi):
            g = wid * gpw + gi
            pltpu.sync_copy(tab_hbm.at[g], tab_v)
            pltpu.sync_copy(idx_hbm.at[g], idx_v)

            @pl.loop(0, m // 16)
            def _(i):
                iv = idx_v[pl.ds(i * 16, 16)] * 8
                for c in range(6):
                    out_v[c, pl.ds(i * 16, 16)] = plsc.load_gather(
                        tab_v, [iv + c])

            pltpu.sync_copy(out_v, out_hbm.at[g])

    return k(tab, idx)


def kernel(point_seq):
    b, t, n, d = point_seq.shape
    nseg = t // _STRIDE
    nc = nseg * b
    half = n // 2
    perm = jax.random.permutation(jax.random.key(42), n)
    perm_a, perm_b = perm[:half], perm[half:]
    inv_perm = jnp.argsort(perm)

    ps_t = point_seq.transpose(0, 1, 3, 2)
    psr_t = ps_t.reshape(b, nseg, _STRIDE, 3, n)
    keys_f = psr_t.reshape(nc, _STRIDE, 3, n)
    keys_r = psr_t[:, :, ::-1].reshape(nc, _STRIDE, 3, n)

    psr = point_seq.reshape(b, nseg, _STRIDE, n, d)
    first = psr[:, :, 0]
    last = psr[:, :, -1]
    q0_fa = first[:, :, perm_a].reshape(nc, half, 3)
    q0_fb = first[:, :, perm_b].reshape(nc, half, 3)
    q0_bw = last[:, :, perm_a].reshape(nc, half, 3)

    dist_a, idx_a, ox_a = _knn_call(keys_f, q0_fa, _K, True)
    (ox_b,) = _knn_call(keys_f, q0_fb, 1, False)
    dist_w, idx_w, _ = _knn_call(keys_r, q0_bw, _K, True)

    def _seq(x):
        return x.reshape((b, nseg) + x.shape[1:]).reshape((b, t) + x.shape[2:])

    def _seq_flip(x):
        tail = x.shape[2:]
        x = x.reshape((b, nseg) + x.shape[1:])
        return x[:, :, ::-1].reshape((b, t) + tail)

    distances = jnp.concatenate([_seq(dist_a), _seq_flip(dist_w)], axis=2)
    idxs = jnp.concatenate([_seq(idx_a), _seq_flip(idx_w)], axis=2)
    ox = jnp.concatenate([_seq(ox_a), _seq(ox_b)], axis=2)
    out_x = jnp.take(ox, inv_perm, axis=2)

    src0 = jnp.concatenate([psr[:, :, :1], psr[:, :, :-1]], axis=2)
    src1 = jnp.concatenate([psr[:, :, 1:], psr[:, :, -1:]], axis=2)
    src0 = src0.reshape(b, t, n, d)
    src1 = src1.reshape(b, t, n, d)
    feats_src = jnp.stack([src0, src1], axis=2)
    pts_src = jnp.broadcast_to(point_seq[:, :, None], (b, t, 2, n, d))
    comb = jnp.concatenate([pts_src, feats_src], axis=-1)
    comb = jnp.pad(comb, ((0, 0),) * 4 + ((0, 8 - 2 * d),))
    tab = comb.reshape(b * t * 2, n * 8)

    idx_flat = idxs.reshape(b * t * 2, half * _K)
    gathered = _sc_gather(tab, idx_flat)
    gathered = gathered.reshape(b, t, 2, 6, half, _K)
    gathered = jnp.moveaxis(gathered, 3, 5)
    patchlet_points = gathered[..., 0:d].reshape(b, t, n, _K, d)
    patchlet_feats = gathered[..., d:2 * d].reshape(b, t, n, _K, d)
    return patchlet_points, patchlet_feats, distances, idxs, out_x


# speedup vs baseline: 32.8774x; 1.2877x over previous
import dataclasses
from functools import partial

import jax
import jax.numpy as jnp
from jax import lax
from jax.experimental import pallas as pl
from jax.experimental.pallas import tpu as pltpu
from jax.experimental.pallas import tpu_sc as plsc

_K = 16
_STRIDE = 8
_ROWS = 128


def _knn_body(n_extract, want_di, keys_ref, q0_ref, *refs):
    if want_di:
        dist_ref, idx_ref, outx_ref, qcur_ref = refs
    else:
        outx_ref, qcur_ref = refs
    s = pl.program_id(2)
    nkeys = keys_ref.shape[3]

    @pl.when(s == 0)
    def _():
        qcur_ref[...] = q0_ref[0]

    q = qcur_ref[...]
    qx, qy, qz = q[:, 0:1], q[:, 1:2], q[:, 2:3]
    kx = keys_ref[0, 0, 0:1, :]
    ky = keys_ref[0, 0, 1:2, :]
    kz = keys_ref[0, 0, 2:3, :]
    dx = qx - kx
    dy = qy - ky
    dz = qz - kz
    dd = (dx * dx + dy * dy) + dz * dz
    iota_f = lax.broadcasted_iota(jnp.int32, dd.shape, 1).astype(jnp.float32)
    big = jnp.float32(2.0 * nkeys)
    ms, mis = [], []
    for e in range(n_extract):
        m = jnp.min(dd, axis=1, keepdims=True)
        mi_f = jnp.min(jnp.where(dd == m, iota_f, big), axis=1,
                       keepdims=True)
        onehot = iota_f == mi_f
        if e == 0:
            ohf = onehot.astype(jnp.float32)
            nx = jnp.sum(ohf * kx, axis=1, keepdims=True)
            ny = jnp.sum(ohf * ky, axis=1, keepdims=True)
            nz = jnp.sum(ohf * kz, axis=1, keepdims=True)
            new_q = jnp.concatenate([nx, ny, nz], axis=1)
            qcur_ref[...] = new_q
            outx_ref[0, 0] = new_q
        if e < n_extract - 1:
            dd = jnp.where(onehot, jnp.float32(jnp.inf), dd)
        ms.append(m)
        mis.append(mi_f)
    if want_di:
        dist_ref[0, 0] = jnp.concatenate(ms, axis=1)
        idx_ref[0, 0] = jnp.concatenate(mis, axis=1).astype(jnp.int32)


def _knn_call(keys, q0, n_extract, want_di, interpret=False):
    c, t, _, n = keys.shape
    r = q0.shape[1]
    grid = (c, r // _ROWS, t)
    out_shapes = []
    out_specs = []
    if want_di:
        out_shapes += [jax.ShapeDtypeStruct((c, t, r, _K), jnp.float32),
                       jax.ShapeDtypeStruct((c, t, r, _K), jnp.int32)]
        out_specs += [pl.BlockSpec((1, 1, _ROWS, _K),
                                   lambda ci, ri, si: (ci, si, ri, 0))] * 2
    out_shapes.append(jax.ShapeDtypeStruct((c, t, r, 3), jnp.float32))
    out_specs.append(pl.BlockSpec((1, 1, _ROWS, 3),
                                  lambda ci, ri, si: (ci, si, ri, 0)))
    return pl.pallas_call(
        partial(_knn_body, n_extract, want_di),
        grid=grid,
        in_specs=[
            pl.BlockSpec((1, 1, 3, n), lambda ci, ri, si: (ci, si, 0, 0)),
            pl.BlockSpec((1, _ROWS, 3), lambda ci, ri, si: (ci, ri, 0)),
        ],
        out_specs=out_specs,
        out_shape=out_shapes,
        scratch_shapes=[pltpu.VMEM((_ROWS, 3), jnp.float32)],
        interpret=interpret,
    )(keys, q0)


def _sc_gather(tab, idx):
    g_tot, m = idx.shape
    nw = 32
    gpw = g_tot // nw
    mesh = plsc.VectorSubcoreMesh(core_axis_name="c", subcore_axis_name="s")
    cp = pltpu.CompilerParams()
    if "needs_layout_passes" in pltpu.CompilerParams.__dataclass_fields__:
        cp = dataclasses.replace(cp, needs_layout_passes=False)

    @partial(pl.kernel,
             out_type=jax.ShapeDtypeStruct((g_tot, 6, m), jnp.float32),
             mesh=mesh,
             scratch_types=[pltpu.VMEM((tab.shape[1],), jnp.float32),
                            pltpu.VMEM((m,), jnp.int32),
                            pltpu.VMEM((6, m), jnp.float32)],
             compiler_params=cp)
    def k(tab_hbm, idx_hbm, out_hbm, tab_v, idx_v, out_v):
        wid = lax.axis_index("s") * 2 + lax.axis_index("c")

        @pl.loop(0, gpw)
        def _(gi):
            g = wid * gpw + gi
            pltpu.sync_copy(tab_hbm.at[g], tab_v)
            pltpu.sync_copy(idx_hbm.at[g], idx_v)

            @pl.loop(0, m // 16)
            def _(i):
                iv = idx_v[pl.ds(i * 16, 16)] * 8
                for c in range(6):
                    out_v[c, pl.ds(i * 16, 16)] = plsc.load_gather(
                        tab_v, [iv + c])

            pltpu.sync_copy(out_v, out_hbm.at[g])

    return k(tab, idx)


def kernel(point_seq):
    b, t, n, d = point_seq.shape
    nseg = t // _STRIDE
    nc = nseg * b
    half = n // 2
    perm = jax.random.permutation(jax.random.key(42), n)
    perm_a, perm_b = perm[:half], perm[half:]
    inv_perm = jnp.argsort(perm)

    ps_t = point_seq.transpose(0, 1, 3, 2)
    psr_t = ps_t.reshape(b, nseg, _STRIDE, 3, n)
    keys_f = psr_t.reshape(nc, _STRIDE, 3, n)
    keys_r = psr_t[:, :, ::-1].reshape(nc, _STRIDE, 3, n)

    psr = point_seq.reshape(b, nseg, _STRIDE, n, d)
    first = psr[:, :, 0]
    last = psr[:, :, -1]
    q0_fa = first[:, :, perm_a].reshape(nc, half, 3)
    q0_fb = first[:, :, perm_b].reshape(nc, half, 3)
    q0_bw = last[:, :, perm_a].reshape(nc, half, 3)

    dist_a, idx_a, ox_a = _knn_call(keys_f, q0_fa, _K, True)
    (ox_b,) = _knn_call(keys_f, q0_fb, 1, False)
    dist_w, idx_w, _ = _knn_call(keys_r, q0_bw, _K, True)

    def _seq(x):
        return x.reshape((b, nseg) + x.shape[1:]).reshape((b, t) + x.shape[2:])

    def _seq_flip(x):
        tail = x.shape[2:]
        x = x.reshape((b, nseg) + x.shape[1:])
        return x[:, :, ::-1].reshape((b, t) + tail)

    distances = jnp.concatenate([_seq(dist_a), _seq_flip(dist_w)], axis=2)
    idxs = jnp.concatenate([_seq(idx_a), _seq_flip(idx_w)], axis=2)
    ox = jnp.concatenate([_seq(ox_a), _seq(ox_b)], axis=2)
    out_x = jnp.take(ox, inv_perm, axis=2)

    src0 = jnp.concatenate([psr[:, :, :1], psr[:, :, :-1]], axis=2)
    src1 = jnp.concatenate([psr[:, :, 1:], psr[:, :, -1:]], axis=2)
    src0 = src0.reshape(b, t, n, d)
    src1 = src1.reshape(b, t, n, d)
    feats_src = jnp.stack([src0, src1], axis=2)
    pts_src = jnp.broadcast_to(point_seq[:, :, None], (b, t, 2, n, d))
    comb = jnp.concatenate([pts_src, feats_src], axis=-1)
    comb = jnp.pad(comb, ((0, 0),) * 4 + ((0, 8 - 2 * d),))
    tab = comb.reshape(b * t * 2, n * 8)

    idx_flat = idxs.reshape(b * t * 2, half * _K)
    gathered = _sc_gather(tab, idx_flat)
    gathered = gathered.reshape(b, t, 2, 6, half, _K)
    gathered = jnp.moveaxis(gathered, 3, 5)
    patchlet_points = gathered[..., 0:d].reshape(b, t, n, _K, d)
    patchlet_feats = gathered[..., d:2 * d].reshape(b, t, n, _K, d)
    return patchlet_points, patchlet_feats, distances, idxs, out_x


# speedup vs baseline: 43.4054x; 1.3202x over previous
import dataclasses
from functools import partial

import jax
import jax.numpy as jnp
from jax import lax
from jax.experimental import pallas as pl
from jax.experimental.pallas import tpu as pltpu
from jax.experimental.pallas import tpu_sc as plsc

_K = 16
_STRIDE = 8
_ROWS = 256


def _knn_body(n_extract, want_di, keys_ref, q0_ref, *refs):
    if want_di:
        dist_ref, idx_ref, outx_ref, qcur_ref = refs
    else:
        outx_ref, qcur_ref = refs
    s = pl.program_id(2)
    nkeys = keys_ref.shape[3]

    @pl.when(s == 0)
    def _():
        qcur_ref[...] = q0_ref[0]

    q = qcur_ref[...]
    qx, qy, qz = q[:, 0:1], q[:, 1:2], q[:, 2:3]
    kx = keys_ref[0, 0, 0:1, :]
    ky = keys_ref[0, 0, 1:2, :]
    kz = keys_ref[0, 0, 2:3, :]
    dx = qx - kx
    dy = qy - ky
    dz = qz - kz
    dd = (dx * dx + dy * dy) + dz * dz
    iota_f = lax.broadcasted_iota(jnp.int32, dd.shape, 1).astype(jnp.float32)
    big = jnp.float32(2.0 * nkeys)
    ms, mis = [], []
    for e in range(n_extract):
        m = jnp.min(dd, axis=1, keepdims=True)
        mi_f = jnp.min(jnp.where(dd == m, iota_f, big), axis=1,
                       keepdims=True)
        onehot = iota_f == mi_f
        if e == 0:
            ohf = onehot.astype(jnp.float32)
            nx = jnp.sum(ohf * kx, axis=1, keepdims=True)
            ny = jnp.sum(ohf * ky, axis=1, keepdims=True)
            nz = jnp.sum(ohf * kz, axis=1, keepdims=True)
            new_q = jnp.concatenate([nx, ny, nz], axis=1)
            qcur_ref[...] = new_q
            outx_ref[0, 0] = new_q
        if e < n_extract - 1:
            dd = jnp.where(onehot, jnp.float32(jnp.inf), dd)
        ms.append(m)
        mis.append(mi_f)
    if want_di:
        dist_ref[0, 0] = jnp.concatenate(ms, axis=1)
        idx_ref[0, 0] = jnp.concatenate(mis, axis=1).astype(jnp.int32)


def _knn_call(keys, q0, n_extract, want_di, interpret=False):
    c, t, _, n = keys.shape
    r = q0.shape[1]
    grid = (c, r // _ROWS, t)
    out_shapes = []
    out_specs = []
    if want_di:
        out_shapes += [jax.ShapeDtypeStruct((c, t, r, _K), jnp.float32),
                       jax.ShapeDtypeStruct((c, t, r, _K), jnp.int32)]
        out_specs += [pl.BlockSpec((1, 1, _ROWS, _K),
                                   lambda ci, ri, si: (ci, si, ri, 0))] * 2
    out_shapes.append(jax.ShapeDtypeStruct((c, t, r, 3), jnp.float32))
    out_specs.append(pl.BlockSpec((1, 1, _ROWS, 3),
                                  lambda ci, ri, si: (ci, si, ri, 0)))
    return pl.pallas_call(
        partial(_knn_body, n_extract, want_di),
        grid=grid,
        in_specs=[
            pl.BlockSpec((1, 1, 3, n), lambda ci, ri, si: (ci, si, 0, 0)),
            pl.BlockSpec((1, _ROWS, 3), lambda ci, ri, si: (ci, ri, 0)),
        ],
        out_specs=out_specs,
        out_shape=out_shapes,
        scratch_shapes=[pltpu.VMEM((_ROWS, 3), jnp.float32)],
        interpret=interpret,
    )(keys, q0)


def _sc_gather(tab, idx):
    g_tot, m = idx.shape
    nw = 32
    gpw = g_tot // nw
    mesh = plsc.VectorSubcoreMesh(core_axis_name="c", subcore_axis_name="s")
    cp = pltpu.CompilerParams()
    if "needs_layout_passes" in pltpu.CompilerParams.__dataclass_fields__:
        cp = dataclasses.replace(cp, needs_layout_passes=False)

    @partial(pl.kernel,
             out_type=jax.ShapeDtypeStruct((g_tot, 6, m), jnp.float32),
             mesh=mesh,
             scratch_types=[pltpu.VMEM((tab.shape[1],), jnp.float32),
                            pltpu.VMEM((m,), jnp.int32),
                            pltpu.VMEM((6, m), jnp.float32)],
             compiler_params=cp)
    def k(tab_hbm, idx_hbm, out_hbm, tab_v, idx_v, out_v):
        wid = lax.axis_index("s") * 2 + lax.axis_index("c")

        @pl.loop(0, gpw)
        def _(gi):
            g = wid * gpw + gi
            pltpu.sync_copy(tab_hbm.at[g], tab_v)
            pltpu.sync_copy(idx_hbm.at[g], idx_v)

            @pl.loop(0, m // 16)
            def _(i):
                iv = idx_v[pl.ds(i * 16, 16)] * 8
                for c in range(6):
                    out_v[c, pl.ds(i * 16, 16)] = plsc.load_gather(
                        tab_v, [iv + c])

            pltpu.sync_copy(out_v, out_hbm.at[g])

    return k(tab, idx)


def kernel(point_seq):
    b, t, n, d = point_seq.shape
    nseg = t // _STRIDE
    nc = nseg * b
    half = n // 2
    perm = jax.random.permutation(jax.random.key(42), n)
    perm_a, perm_b = perm[:half], perm[half:]
    inv_perm = jnp.argsort(perm)

    ps_t = point_seq.transpose(0, 1, 3, 2)
    psr_t = ps_t.reshape(b, nseg, _STRIDE, 3, n)
    keys_f = psr_t.reshape(nc, _STRIDE, 3, n)
    keys_r = psr_t[:, :, ::-1].reshape(nc, _STRIDE, 3, n)

    psr = point_seq.reshape(b, nseg, _STRIDE, n, d)
    first = psr[:, :, 0]
    last = psr[:, :, -1]
    q0_fa = first[:, :, perm_a].reshape(nc, half, 3)
    q0_fb = first[:, :, perm_b].reshape(nc, half, 3)
    q0_bw = last[:, :, perm_a].reshape(nc, half, 3)

    dist_a, idx_a, ox_a = _knn_call(keys_f, q0_fa, _K, True)
    (ox_b,) = _knn_call(keys_f, q0_fb, 1, False)
    dist_w, idx_w, _ = _knn_call(keys_r, q0_bw, _K, True)

    def _seq(x):
        return x.reshape((b, nseg) + x.shape[1:]).reshape((b, t) + x.shape[2:])

    def _seq_flip(x):
        tail = x.shape[2:]
        x = x.reshape((b, nseg) + x.shape[1:])
        return x[:, :, ::-1].reshape((b, t) + tail)

    distances = jnp.concatenate([_seq(dist_a), _seq_flip(dist_w)], axis=2)
    idxs = jnp.concatenate([_seq(idx_a), _seq_flip(idx_w)], axis=2)
    ox = jnp.concatenate([_seq(ox_a), _seq(ox_b)], axis=2)
    out_x = jnp.take(ox, inv_perm, axis=2)

    src0 = jnp.concatenate([psr[:, :, :1], psr[:, :, :-1]], axis=2)
    src1 = jnp.concatenate([psr[:, :, 1:], psr[:, :, -1:]], axis=2)
    src0 = src0.reshape(b, t, n, d)
    src1 = src1.reshape(b, t, n, d)
    feats_src = jnp.stack([src0, src1], axis=2)
    pts_src = jnp.broadcast_to(point_seq[:, :, None], (b, t, 2, n, d))
    comb = jnp.concatenate([pts_src, feats_src], axis=-1)
    comb = jnp.pad(comb, ((0, 0),) * 4 + ((0, 8 - 2 * d),))
    tab = comb.reshape(b * t * 2, n * 8)

    idx_flat = idxs.reshape(b * t * 2, half * _K)
    gathered = _sc_gather(tab, idx_flat)
    gathered = gathered.reshape(b, t, 2, 6, half, _K)
    gathered = jnp.moveaxis(gathered, 3, 5)
    patchlet_points = gathered[..., 0:d].reshape(b, t, n, _K, d)
    patchlet_feats = gathered[..., d:2 * d].reshape(b, t, n, _K, d)
    return patchlet_points, patchlet_feats, distances, idxs, out_x


# speedup vs baseline: 48.8424x; 1.1253x over previous
import dataclasses
from functools import partial

import jax
import jax.numpy as jnp
from jax import lax
from jax.experimental import pallas as pl
from jax.experimental.pallas import tpu as pltpu
from jax.experimental.pallas import tpu_sc as plsc

_K = 16
_STRIDE = 8
_ROWS = 256


def _knn_body(n_extract, want_di, keys_ref, q0_ref, *refs):
    if want_di:
        dist_ref, idx_ref, outx_ref, qcur_ref = refs
    else:
        outx_ref, qcur_ref = refs
    s = pl.program_id(2)
    nkeys = keys_ref.shape[3]

    @pl.when(s == 0)
    def _():
        qcur_ref[...] = q0_ref[0]

    q = qcur_ref[...]
    qx, qy, qz = q[:, 0:1], q[:, 1:2], q[:, 2:3]
    kx = keys_ref[0, 0, 0:1, :]
    ky = keys_ref[0, 0, 1:2, :]
    kz = keys_ref[0, 0, 2:3, :]
    dx = qx - kx
    dy = qy - ky
    dz = qz - kz
    dd = (dx * dx + dy * dy) + dz * dz
    iota_f = lax.broadcasted_iota(jnp.int32, dd.shape, 1).astype(jnp.float32)
    big = jnp.float32(2.0 * nkeys)
    ms, mis = [], []
    for e in range(n_extract):
        m = jnp.min(dd, axis=1, keepdims=True)
        mi_f = jnp.min(jnp.where(dd == m, iota_f, big), axis=1,
                       keepdims=True)
        onehot = iota_f == mi_f
        if e == 0:
            ohf = onehot.astype(jnp.float32)
            nx = jnp.sum(ohf * kx, axis=1, keepdims=True)
            ny = jnp.sum(ohf * ky, axis=1, keepdims=True)
            nz = jnp.sum(ohf * kz, axis=1, keepdims=True)
            new_q = jnp.concatenate([nx, ny, nz], axis=1)
            qcur_ref[...] = new_q
            outx_ref[0, 0] = new_q
        if e < n_extract - 1:
            dd = jnp.where(onehot, jnp.float32(jnp.inf), dd)
        ms.append(m)
        mis.append(mi_f)
    if want_di:
        dist_ref[0, 0] = jnp.concatenate(ms, axis=1)
        idx_ref[0, 0] = jnp.concatenate(mis, axis=1).astype(jnp.int32)


def _knn_call(keys, q0, n_extract, want_di, interpret=False):
    c, t, _, n = keys.shape
    r = q0.shape[1]
    grid = (c, r // _ROWS, t)
    out_shapes = []
    out_specs = []
    if want_di:
        out_shapes += [jax.ShapeDtypeStruct((c, t, r, _K), jnp.float32),
                       jax.ShapeDtypeStruct((c, t, r, _K), jnp.int32)]
        out_specs += [pl.BlockSpec((1, 1, _ROWS, _K),
                                   lambda ci, ri, si: (ci, si, ri, 0))] * 2
    out_shapes.append(jax.ShapeDtypeStruct((c, t, r, 3), jnp.float32))
    out_specs.append(pl.BlockSpec((1, 1, _ROWS, 3),
                                  lambda ci, ri, si: (ci, si, ri, 0)))
    return pl.pallas_call(
        partial(_knn_body, n_extract, want_di),
        grid=grid,
        in_specs=[
            pl.BlockSpec((1, 1, 3, n), lambda ci, ri, si: (ci, si, 0, 0)),
            pl.BlockSpec((1, _ROWS, 3), lambda ci, ri, si: (ci, ri, 0)),
        ],
        out_specs=out_specs,
        out_shape=out_shapes,
        scratch_shapes=[pltpu.VMEM((_ROWS, 3), jnp.float32)],
        interpret=interpret,
    )(keys, q0)


def _sc_gather(tab, idx):
    g_tot, m = idx.shape
    nw = 32
    gpw = g_tot // nw
    mesh = plsc.VectorSubcoreMesh(core_axis_name="c", subcore_axis_name="s")
    cp = pltpu.CompilerParams()
    if "needs_layout_passes" in pltpu.CompilerParams.__dataclass_fields__:
        cp = dataclasses.replace(cp, needs_layout_passes=False)

    @partial(pl.kernel,
             out_type=[jax.ShapeDtypeStruct((g_tot, 3 * m), jnp.float32),
                       jax.ShapeDtypeStruct((g_tot, 3 * m), jnp.float32)],
             mesh=mesh,
             scratch_types=[pltpu.VMEM((tab.shape[1],), jnp.float32),
                            pltpu.VMEM((m,), jnp.int32),
                            pltpu.VMEM((3 * m,), jnp.float32),
                            pltpu.VMEM((3 * m,), jnp.float32)],
             compiler_params=cp)
    def k(tab_hbm, idx_hbm, pp_hbm, pf_hbm, tab_v, idx_v, pp_v, pf_v):
        wid = lax.axis_index("s") * 2 + lax.axis_index("c")
        lane = lax.iota(jnp.int32, 16)

        @pl.loop(0, gpw)
        def _(gi):
            g = wid * gpw + gi
            pltpu.sync_copy(tab_hbm.at[g], tab_v)
            pltpu.sync_copy(idx_hbm.at[g], idx_v)

            @pl.loop(0, m // 16)
            def _(i):
                iv = idx_v[pl.ds(i * 16, 16)] * 8
                pos = lane * 3 + i * 48
                for c in range(3):
                    plsc.store_scatter(pp_v, [pos + c],
                                       plsc.load_gather(tab_v, [iv + c]))
                    plsc.store_scatter(pf_v, [pos + c],
                                       plsc.load_gather(tab_v, [iv + (c + 3)]))

            pltpu.sync_copy(pp_v, pp_hbm.at[g])
            pltpu.sync_copy(pf_v, pf_hbm.at[g])

    return k(tab, idx)


def kernel(point_seq):
    b, t, n, d = point_seq.shape
    nseg = t // _STRIDE
    nc = nseg * b
    half = n // 2
    perm = jax.random.permutation(jax.random.key(42), n)
    perm_a, perm_b = perm[:half], perm[half:]
    inv_perm = jnp.argsort(perm)

    ps_t = point_seq.transpose(0, 1, 3, 2)
    psr_t = ps_t.reshape(b, nseg, _STRIDE, 3, n)
    keys_f = psr_t.reshape(nc, _STRIDE, 3, n)
    keys_r = psr_t[:, :, ::-1].reshape(nc, _STRIDE, 3, n)

    psr = point_seq.reshape(b, nseg, _STRIDE, n, d)
    first = psr[:, :, 0]
    last = psr[:, :, -1]
    q0_fa = first[:, :, perm_a].reshape(nc, half, 3)
    q0_fb = first[:, :, perm_b].reshape(nc, half, 3)
    q0_bw = last[:, :, perm_a].reshape(nc, half, 3)

    dist_a, idx_a, ox_a = _knn_call(keys_f, q0_fa, _K, True)
    (ox_b,) = _knn_call(keys_f, q0_fb, 1, False)
    dist_w, idx_w, _ = _knn_call(keys_r, q0_bw, _K, True)

    def _seq(x):
        return x.reshape((b, nseg) + x.shape[1:]).reshape((b, t) + x.shape[2:])

    def _seq_flip(x):
        tail = x.shape[2:]
        x = x.reshape((b, nseg) + x.shape[1:])
        return x[:, :, ::-1].reshape((b, t) + tail)

    distances = jnp.concatenate([_seq(dist_a), _seq_flip(dist_w)], axis=2)
    idxs = jnp.concatenate([_seq(idx_a), _seq_flip(idx_w)], axis=2)
    ox = jnp.concatenate([_seq(ox_a), _seq(ox_b)], axis=2)
    out_x = jnp.take(ox, inv_perm, axis=2)

    src0 = jnp.concatenate([psr[:, :, :1], psr[:, :, :-1]], axis=2)
    src1 = jnp.concatenate([psr[:, :, 1:], psr[:, :, -1:]], axis=2)
    src0 = src0.reshape(b, t, n, d)
    src1 = src1.reshape(b, t, n, d)
    pad = ((0, 0),) * 3 + ((0, 8 - 2 * d),)
    tab0 = jnp.pad(jnp.concatenate([point_seq, src0], -1), pad)
    tab1 = jnp.pad(jnp.concatenate([point_seq, src1], -1), pad)
    m = half * _K
    pp0, pf0 = _sc_gather(tab0.reshape(b * t, n * 8),
                          _seq(idx_a).reshape(b * t, m))
    pp1, pf1 = _sc_gather(tab1.reshape(b * t, n * 8),
                          _seq_flip(idx_w).reshape(b * t, m))

    def _halves(x0, x1):
        return jnp.concatenate([x0.reshape(b, t, half, _K, d),
                                x1.reshape(b, t, half, _K, d)], axis=2)

    patchlet_points = _halves(pp0, pp1)
    patchlet_feats = _halves(pf0, pf1)
    return patchlet_points, patchlet_feats, distances, idxs, out_x


# speedup vs baseline: 49.5688x; 1.0149x over previous
import dataclasses
from functools import partial

import jax
import jax.numpy as jnp
from jax import lax
from jax.experimental import pallas as pl
from jax.experimental.pallas import tpu as pltpu
from jax.experimental.pallas import tpu_sc as plsc

_K = 16
_STRIDE = 8
_ROWS = 256


def _knn_body(n_extract, want_di, keys_ref, q0_ref, *refs):
    if want_di:
        dist_ref, idx_ref, outx_ref, qcur_ref = refs
    else:
        outx_ref, qcur_ref = refs
    s = pl.program_id(2)
    nkeys = keys_ref.shape[3]

    @pl.when(s == 0)
    def _():
        qcur_ref[...] = q0_ref[0]

    q = qcur_ref[...]
    qx, qy, qz = q[:, 0:1], q[:, 1:2], q[:, 2:3]
    kx = keys_ref[0, 0, 0:1, :]
    ky = keys_ref[0, 0, 1:2, :]
    kz = keys_ref[0, 0, 2:3, :]
    dx = qx - kx
    dy = qy - ky
    dz = qz - kz
    dd = (dx * dx + dy * dy) + dz * dz
    iota_f = lax.broadcasted_iota(jnp.int32, dd.shape, 1).astype(jnp.float32)
    big = jnp.float32(2.0 * nkeys)
    ms, mis = [], []
    for e in range(n_extract):
        m = jnp.min(dd, axis=1, keepdims=True)
        mi_f = jnp.min(jnp.where(dd == m, iota_f, big), axis=1,
                       keepdims=True)
        onehot = iota_f == mi_f
        if e == 0:
            ohf = onehot.astype(jnp.float32)
            nx = jnp.sum(ohf * kx, axis=1, keepdims=True)
            ny = jnp.sum(ohf * ky, axis=1, keepdims=True)
            nz = jnp.sum(ohf * kz, axis=1, keepdims=True)
            new_q = jnp.concatenate([nx, ny, nz], axis=1)
            qcur_ref[...] = new_q
            outx_ref[0, 0] = new_q
        if e < n_extract - 1:
            dd = jnp.where(onehot, jnp.float32(jnp.inf), dd)
        ms.append(m)
        mis.append(mi_f)
    if want_di:
        dist_ref[0, 0] = jnp.concatenate(ms, axis=1)
        idx_ref[0, 0] = jnp.concatenate(mis, axis=1).astype(jnp.int32)


def _knn_call(keys, q0, n_extract, want_di, flip_t=False, interpret=False):
    c, t, _, n = keys.shape
    r = q0.shape[1]
    grid = (c, r // _ROWS, t)
    if flip_t:
        omap = lambda ci, ri, si: (ci, t - 1 - si, ri, 0)
    else:
        omap = lambda ci, ri, si: (ci, si, ri, 0)
    out_shapes = []
    out_specs = []
    if want_di:
        out_shapes += [jax.ShapeDtypeStruct((c, t, r, _K), jnp.float32),
                       jax.ShapeDtypeStruct((c, t, r, _K), jnp.int32)]
        out_specs += [pl.BlockSpec((1, 1, _ROWS, _K), omap)] * 2
    out_shapes.append(jax.ShapeDtypeStruct((c, t, r, 3), jnp.float32))
    out_specs.append(pl.BlockSpec((1, 1, _ROWS, 3), omap))
    return pl.pallas_call(
        partial(_knn_body, n_extract, want_di),
        grid=grid,
        in_specs=[
            pl.BlockSpec((1, 1, 3, n), lambda ci, ri, si: (ci, si, 0, 0)),
            pl.BlockSpec((1, _ROWS, 3), lambda ci, ri, si: (ci, ri, 0)),
        ],
        out_specs=out_specs,
        out_shape=out_shapes,
        scratch_shapes=[pltpu.VMEM((_ROWS, 3), jnp.float32)],
        interpret=interpret,
    )(keys, q0)


def _sc_gather(tab, idx):
    g_tot, m = idx.shape
    nw = 32
    gpw = g_tot // nw
    mesh = plsc.VectorSubcoreMesh(core_axis_name="c", subcore_axis_name="s")
    cp = pltpu.CompilerParams()
    if "needs_layout_passes" in pltpu.CompilerParams.__dataclass_fields__:
        cp = dataclasses.replace(cp, needs_layout_passes=False)

    @partial(pl.kernel,
             out_type=[jax.ShapeDtypeStruct((g_tot, 3 * m), jnp.float32),
                       jax.ShapeDtypeStruct((g_tot, 3 * m), jnp.float32)],
             mesh=mesh,
             scratch_types=[pltpu.VMEM((tab.shape[1],), jnp.float32),
                            pltpu.VMEM((m,), jnp.int32),
                            pltpu.VMEM((3 * m,), jnp.float32),
                            pltpu.VMEM((3 * m,), jnp.float32)],
             compiler_params=cp)
    def k(tab_hbm, idx_hbm, pp_hbm, pf_hbm, tab_v, idx_v, pp_v, pf_v):
        wid = lax.axis_index("s") * 2 + lax.axis_index("c")
        lane = lax.iota(jnp.int32, 16)

        @pl.loop(0, gpw)
        def _(gi):
            g = wid * gpw + gi
            pltpu.sync_copy(tab_hbm.at[g], tab_v)
            pltpu.sync_copy(idx_hbm.at[g], idx_v)

            @pl.loop(0, m // 16)
            def _(i):
                iv = idx_v[pl.ds(i * 16, 16)] * 8
                pos = lane * 3 + i * 48
                for c in range(3):
                    plsc.store_scatter(pp_v, [pos + c],
                                       plsc.load_gather(tab_v, [iv + c]))
                    plsc.store_scatter(pf_v, [pos + c],
                                       plsc.load_gather(tab_v, [iv + (c + 3)]))

            pltpu.sync_copy(pp_v, pp_hbm.at[g])
            pltpu.sync_copy(pf_v, pf_hbm.at[g])

    return k(tab, idx)


def kernel(point_seq):
    b, t, n, d = point_seq.shape
    nseg = t // _STRIDE
    nc = nseg * b
    half = n // 2
    perm = jax.random.permutation(jax.random.key(42), n)
    perm_a, perm_b = perm[:half], perm[half:]
    inv_perm = jnp.argsort(perm)

    ps_t = point_seq.transpose(0, 1, 3, 2)
    psr_t = ps_t.reshape(b, nseg, _STRIDE, 3, n)
    keys_f = psr_t.reshape(nc, _STRIDE, 3, n)
    keys_r = psr_t[:, :, ::-1].reshape(nc, _STRIDE, 3, n)

    psr = point_seq.reshape(b, nseg, _STRIDE, n, d)
    first = psr[:, :, 0]
    last = psr[:, :, -1]
    q0_fa = first[:, :, perm_a].reshape(nc, half, 3)
    q0_fb = first[:, :, perm_b].reshape(nc, half, 3)
    q0_bw = last[:, :, perm_a].reshape(nc, half, 3)

    dist_a, idx_a, ox_a = _knn_call(keys_f, q0_fa, _K, True)
    (ox_b,) = _knn_call(keys_f, q0_fb, 1, False)
    dist_w, idx_w, _ = _knn_call(keys_r, q0_bw, _K, True, flip_t=True)

    def _seq(x):
        return x.reshape((b, nseg) + x.shape[1:]).reshape((b, t) + x.shape[2:])

    distances = jnp.concatenate([_seq(dist_a), _seq(dist_w)], axis=2)
    idxs = jnp.concatenate([_seq(idx_a), _seq(idx_w)], axis=2)
    ox = jnp.concatenate([_seq(ox_a), _seq(ox_b)], axis=2)
    out_x = jnp.take(ox, inv_perm, axis=2)

    src0 = jnp.concatenate([psr[:, :, :1], psr[:, :, :-1]], axis=2)
    src1 = jnp.concatenate([psr[:, :, 1:], psr[:, :, -1:]], axis=2)
    src0 = src0.reshape(b, t, n, d)
    src1 = src1.reshape(b, t, n, d)
    pad = ((0, 0),) * 3 + ((0, 8 - 2 * d),)
    tab0 = jnp.pad(jnp.concatenate([point_seq, src0], -1), pad)
    tab1 = jnp.pad(jnp.concatenate([point_seq, src1], -1), pad)
    m = half * _K
    pp0, pf0 = _sc_gather(tab0.reshape(b * t, n * 8),
                          _seq(idx_a).reshape(b * t, m))
    pp1, pf1 = _sc_gather(tab1.reshape(b * t, n * 8),
                          _seq(idx_w).reshape(b * t, m))

    def _halves(x0, x1):
        return jnp.concatenate([x0.reshape(b, t, half, _K, d),
                                x1.reshape(b, t, half, _K, d)], axis=2)

    patchlet_points = _halves(pp0, pp1)
    patchlet_feats = _halves(pf0, pf1)
    return patchlet_points, patchlet_feats, distances, idxs, out_x
